# Initial kernel scaffold; baseline (speedup 1.0000x reference)
import jax
import jax.numpy as jnp
from jax.experimental import pallas as pl


def kernel(x, edge_idx, W1, b1, W2, b2, W3, b3):
    raise NotImplementedError("write your pallas kernel here")


# speedup vs baseline: 10.7087x; 10.7087x over previous
import functools

import jax
import jax.numpy as jnp
from jax import lax
from jax.experimental import pallas as pl
from jax.experimental.pallas import tpu as pltpu
from jax.experimental.pallas import tpu_sc as plsc

N = 10000
NP = 10240
E = 320000
NC = 2
NS = 16
NW = NC * NS
EPT = E // NW
K = 80
NCHUNK = EPT // K
RPT = NP // NS
ZR = 64
BLK = 256
GRID = NP // BLK

_mesh = plsc.VectorSubcoreMesh(core_axis_name="c", subcore_axis_name="s")


def _zero_vmem2(ref, rows, cols):
    def body(r, _):
        for j in range(cols // 16):
            ref[r, pl.ds(16 * j, 16)] = jnp.zeros((16,), jnp.float32)
        return 0

    lax.fori_loop(0, rows, body, 0)


@functools.partial(
    pl.kernel,
    out_type=jax.ShapeDtypeStruct((NC, NP), jnp.float32),
    mesh=_mesh,
    scratch_types=[
        pltpu.VMEM((K,), jnp.int32),
        pltpu.VMEM((K,), jnp.float32),
        pltpu.VMEM((RPT,), jnp.float32),
        pltpu.VMEM_SHARED((NP,), jnp.float32),
    ],
)
def _deg_kernel(dst_hbm, out_hbm, idx_v, ones_v, zer_v, deg_sp):
    c = lax.axis_index("c")
    s = lax.axis_index("s")

    for j in range(K // 16):
        ones_v[pl.ds(16 * j, 16)] = jnp.ones((16,), jnp.float32)

    def zbody(r, _):
        zer_v[pl.ds(16 * r, 16)] = jnp.zeros((16,), jnp.float32)
        return 0

    lax.fori_loop(0, RPT // 16, zbody, 0)
    pltpu.sync_copy(zer_v, deg_sp.at[pl.ds(RPT * s, RPT)])
    plsc.subcore_barrier()

    base = (s * NC + c) * EPT

    def body(j, _):
        pltpu.sync_copy(dst_hbm.at[pl.ds(base + j * K, K)], idx_v)
        pltpu.sync_copy(ones_v, deg_sp.at[idx_v], add=True)
        return 0

    lax.fori_loop(0, NCHUNK, body, 0)
    plsc.subcore_barrier()
    pltpu.sync_copy(deg_sp.at[pl.ds(RPT * s, RPT)],
                    out_hbm.at[c, pl.ds(RPT * s, RPT)])


def _make_scatter_kernel(F):
    @functools.partial(
        pl.kernel,
        out_type=jax.ShapeDtypeStruct((NC, NP, F), jnp.float32),
        mesh=_mesh,
        scratch_types=[
            pltpu.VMEM((K,), jnp.int32),
            pltpu.VMEM((K,), jnp.int32),
            pltpu.VMEM((K, F), jnp.float32),
            pltpu.VMEM((ZR, F), jnp.float32),
            pltpu.VMEM_SHARED((NP, F), jnp.float32),
        ],
    )
    def _scatter_kernel(hs_hbm, src_hbm, dst_hbm, out_hbm,
                        src_v, dst_v, buf, zbuf, acc_sp):
        c = lax.axis_index("c")
        s = lax.axis_index("s")

        _zero_vmem2(zbuf, ZR, F)

        def zcopy(j, _):
            pltpu.sync_copy(zbuf, acc_sp.at[pl.ds(RPT * s + ZR * j, ZR)])
            return 0

        lax.fori_loop(0, RPT // ZR, zcopy, 0)
        plsc.subcore_barrier()

        base = c * (E // NC) + s * EPT

        def body(j, _):
            off = base + j * K
            pltpu.sync_copy(src_hbm.at[pl.ds(off, K)], src_v)
            pltpu.sync_copy(dst_hbm.at[pl.ds(off, K)], dst_v)
            pltpu.sync_copy(hs_hbm.at[src_v], buf)
            pltpu.sync_copy(buf, acc_sp.at[dst_v], add=True)
            return 0

        lax.fori_loop(0, NCHUNK, body, 0)
        plsc.subcore_barrier()
        pltpu.sync_copy(acc_sp.at[pl.ds(RPT * s, RPT)],
                        out_hbm.at[c, pl.ds(RPT * s, RPT)])

    return _scatter_kernel


_scatter128 = _make_scatter_kernel(128)


def _tc1_body(x_ref, w_ref, degp_ref, hs_ref, dinv_ref):
    i = pl.program_id(0)
    deg = degp_ref[0, pl.ds(i * BLK, BLK)] + degp_ref[1, pl.ds(i * BLK, BLK)] + 1.0
    dinv = lax.rsqrt(deg)
    h = jnp.dot(x_ref[...], w_ref[...], preferred_element_type=jnp.float32)
    hs_ref[...] = h * dinv[:, None]
    dinv_ref[0, pl.ds(i * BLK, BLK)] = dinv


def _tc1(x_pad, W, degp):
    return pl.pallas_call(
        _tc1_body,
        grid=(GRID,),
        in_specs=[
            pl.BlockSpec((BLK, 128), lambda i: (i, 0)),
            pl.BlockSpec((128, 128), lambda i: (0, 0)),
            pl.BlockSpec((NC, NP), lambda i: (0, 0)),
        ],
        out_specs=[
            pl.BlockSpec((BLK, 128), lambda i: (i, 0)),
            pl.BlockSpec((1, NP), lambda i: (0, 0)),
        ],
        out_shape=[
            jax.ShapeDtypeStruct((NP, 128), jnp.float32),
            jax.ShapeDtypeStruct((1, NP), jnp.float32),
        ],
    )(x_pad, W, degp)


def _make_tc_mid(F_in, F_out):
    def body(accp_ref, hs_ref, dinv_ref, b_ref, w_ref, out_ref):
        i = pl.program_id(0)
        dv = dinv_ref[0, pl.ds(i * BLK, BLK)]
        tot = accp_ref[0] + accp_ref[1] + hs_ref[...]
        o = jnp.maximum(dv[:, None] * tot + b_ref[0][None, :], 0.0)
        out_ref[...] = jnp.dot(o, w_ref[...],
                               preferred_element_type=jnp.float32) * dv[:, None]

    def run(accp, hs, dinv, b, W):
        return pl.pallas_call(
            body,
            grid=(GRID,),
            in_specs=[
                pl.BlockSpec((NC, BLK, F_in), lambda i: (0, i, 0)),
                pl.BlockSpec((BLK, F_in), lambda i: (i, 0)),
                pl.BlockSpec((1, NP), lambda i: (0, 0)),
                pl.BlockSpec((1, F_in), lambda i: (0, 0)),
                pl.BlockSpec((F_in, F_out), lambda i: (0, 0)),
            ],
            out_specs=pl.BlockSpec((BLK, F_out), lambda i: (i, 0)),
            out_shape=jax.ShapeDtypeStruct((NP, F_out), jnp.float32),
        )(accp, hs, dinv, b, W)

    return run


_tc_mid_128 = _make_tc_mid(128, 128)


def _tc_fin_body(accp_ref, hs_ref, dinv_ref, b_ref, out_ref):
    i = pl.program_id(0)
    dv = dinv_ref[0, pl.ds(i * BLK, BLK)]
    tot = accp_ref[0] + accp_ref[1] + hs_ref[...]
    o = dv[:, None] * tot + b_ref[0][None, :]
    col = lax.broadcasted_iota(jnp.int32, (BLK, 128), 1)
    valid = col < 40
    xm = jnp.where(valid, o, -jnp.inf)
    m = jnp.max(xm, axis=1)
    e = jnp.exp(xm - m[:, None])
    ssum = jnp.sum(e, axis=1)
    out_ref[...] = o - m[:, None] - jnp.log(ssum)[:, None]


def _tc_fin(accp, hs, dinv, b):
    return pl.pallas_call(
        _tc_fin_body,
        grid=(GRID,),
        in_specs=[
            pl.BlockSpec((NC, BLK, 128), lambda i: (0, i, 0)),
            pl.BlockSpec((BLK, 128), lambda i: (i, 0)),
            pl.BlockSpec((1, NP), lambda i: (0, 0)),
            pl.BlockSpec((1, 128), lambda i: (0, 0)),
        ],
        out_specs=pl.BlockSpec((BLK, 128), lambda i: (i, 0)),
        out_shape=jax.ShapeDtypeStruct((NP, 128), jnp.float32),
    )(accp, hs, dinv, b)


def kernel(x, edge_idx, W1, b1, W2, b2, W3, b3):
    src = edge_idx[0]
    dst = edge_idx[1]
    x_pad = jnp.pad(x, ((0, NP - N), (0, 0)))
    W3p = jnp.pad(W3, ((0, 0), (0, 128 - 40)))
    b1r = b1[None, :]
    b2r = b2[None, :]
    b3r = jnp.pad(b3, (0, 128 - 40))[None, :]

    degp = _deg_kernel(dst)

    hs1, dinv = _tc1(x_pad, W1, degp)
    acc1 = _scatter128(hs1, src, dst)
    hs2 = _tc_mid_128(acc1, hs1, dinv, b1r, W2)
    acc2 = _scatter128(hs2, src, dst)
    hs3 = _tc_mid_128(acc2, hs2, dinv, b2r, W3p)
    acc3 = _scatter128(hs3, src, dst)
    out = _tc_fin(acc3, hs3, dinv, b3r)
    return out[:N, :40]


# speedup vs baseline: 24.1881x; 2.2587x over previous
import functools

import jax
import jax.numpy as jnp
from jax import lax
from jax.experimental import pallas as pl
from jax.experimental.pallas import tpu as pltpu
from jax.experimental.pallas import tpu_sc as plsc

N = 10000
NP = 10240
E = 320000
NC = 2
NS = 16
NW = NC * NS
EPT = E // NW
K = 80
NCHUNK = EPT // K
RPT = NP // NS
ZR = 16
BLK = 256
GRID = NP // BLK

_mesh = plsc.VectorSubcoreMesh(core_axis_name="c", subcore_axis_name="s")


def _zero_vmem2(ref, rows, cols):
    def body(r, _):
        for j in range(cols // 16):
            ref[r, pl.ds(16 * j, 16)] = jnp.zeros((16,), jnp.float32)
        return 0

    lax.fori_loop(0, rows, body, 0)


@functools.partial(
    pl.kernel,
    out_type=jax.ShapeDtypeStruct((NC, NP), jnp.float32),
    mesh=_mesh,
    scratch_types=[
        pltpu.VMEM((NCHUNK, K), jnp.int32),
        pltpu.VMEM((K,), jnp.float32),
        pltpu.VMEM((RPT,), jnp.float32),
        pltpu.VMEM_SHARED((NP,), jnp.float32),
        pltpu.SemaphoreType.DMA,
    ],
)
def _deg_kernel(dst_hbm, out_hbm, idx_v, ones_v, zer_v, deg_sp, dsem):
    c = lax.axis_index("c")
    s = lax.axis_index("s")

    pltpu.sync_copy(dst_hbm.at[c, s], idx_v)
    for j in range(K // 16):
        ones_v[pl.ds(16 * j, 16)] = jnp.ones((16,), jnp.float32)

    def zbody(r, _):
        zer_v[pl.ds(16 * r, 16)] = jnp.zeros((16,), jnp.float32)
        return 0

    lax.fori_loop(0, RPT // 16, zbody, 0)
    pltpu.sync_copy(zer_v, deg_sp.at[pl.ds(RPT * s, RPT)])
    plsc.subcore_barrier()

    GR = 25

    def body(g, _):
        for t in range(GR):
            pltpu.async_copy(ones_v, deg_sp.at[idx_v.at[g * GR + t]], dsem,
                             add=True)
        for t in range(GR):
            pltpu.make_async_copy(ones_v, deg_sp.at[idx_v.at[g * GR + t]],
                                  dsem).wait()
        return 0

    lax.fori_loop(0, NCHUNK // GR, body, 0)
    plsc.subcore_barrier()
    pltpu.sync_copy(deg_sp.at[pl.ds(RPT * s, RPT)],
                    out_hbm.at[c, pl.ds(RPT * s, RPT)])


NBUF = 2


def _make_scatter_kernel(F):
    @functools.partial(
        pl.kernel,
        out_type=jax.ShapeDtypeStruct((NC, NP, F), jnp.float32),
        mesh=_mesh,
        scratch_types=[
            pltpu.VMEM((4, K), jnp.int32),
            pltpu.VMEM((4, K), jnp.int32),
        ] + [pltpu.VMEM((K, F), jnp.float32) for _ in range(NBUF)] + [
            pltpu.VMEM((ZR, F), jnp.float32),
            pltpu.VMEM_SHARED((NP, F), jnp.float32),
            pltpu.SemaphoreType.DMA((4,)),
            pltpu.SemaphoreType.DMA((NBUF,)),
            pltpu.SemaphoreType.DMA,
        ],
    )
    def _scatter_kernel(hs_hbm, src_hbm, dst_hbm, out_hbm,
                        src_v, dst_v, b0, b1,
                        zbuf, acc_sp, isem, gsem, ssem):
        c = lax.axis_index("c")
        s = lax.axis_index("s")
        bufs = [b0, b1]

        def fire_idx(j):
            sl = lax.rem(j, 4)
            pltpu.async_copy(src_hbm.at[c, s, j], src_v.at[sl], isem.at[sl])
            pltpu.async_copy(dst_hbm.at[c, s, j], dst_v.at[sl], isem.at[sl])

        def wait_idx(j):
            sl = lax.rem(j, 4)
            pltpu.make_async_copy(src_hbm.at[c, s, j], src_v.at[sl],
                                  isem.at[sl]).wait()
            pltpu.make_async_copy(dst_hbm.at[c, s, j], dst_v.at[sl],
                                  isem.at[sl]).wait()

        for j in range(4):
            fire_idx(j)
        for b in range(NBUF):
            wait_idx(b)
            pltpu.async_copy(hs_hbm.at[src_v.at[b]], bufs[b], gsem.at[b])

        _zero_vmem2(zbuf, ZR, F)

        def zcopy(j, _):
            pltpu.sync_copy(zbuf, acc_sp.at[pl.ds(RPT * s + ZR * j, ZR)])
            return 0

        lax.fori_loop(0, RPT // ZR, zcopy, 0)
        plsc.subcore_barrier()

        def step(j, b):
            sl = lax.rem(j, 4)
            pltpu.make_async_copy(hs_hbm.at[src_v.at[sl]], bufs[b],
                                  gsem.at[b]).wait()
            pltpu.async_copy(bufs[b], acc_sp.at[dst_v.at[sl]], ssem,
                             add=True).wait()

            @pl.when(j + 4 < NCHUNK)
            def _():
                fire_idx(j + 4)

            @pl.when(j + NBUF < NCHUNK)
            def _():
                wait_idx(j + NBUF)
                pltpu.async_copy(hs_hbm.at[src_v.at[lax.rem(j + NBUF, 4)]],
                                 bufs[b], gsem.at[b])

        def body(jj, _):
            for b in range(NBUF):
                step(jj * NBUF + b, b)
            return 0

        lax.fori_loop(0, NCHUNK // NBUF, body, 0)
        step(NCHUNK - 1, 0)
        plsc.subcore_barrier()
        pltpu.sync_copy(acc_sp.at[pl.ds(RPT * s, RPT)],
                        out_hbm.at[c, pl.ds(RPT * s, RPT)])

    return _scatter_kernel


_scatter128 = _make_scatter_kernel(128)


def _tc1_body(x_ref, w_ref, degp_ref, hs_ref, dinv_ref):
    i = pl.program_id(0)
    deg = degp_ref[0, pl.ds(i * BLK, BLK)] + degp_ref[1, pl.ds(i * BLK, BLK)] + 1.0
    dinv = lax.rsqrt(deg)
    h = jnp.dot(x_ref[...], w_ref[...], preferred_element_type=jnp.float32)
    hs_ref[...] = h * dinv[:, None]
    dinv_ref[0, pl.ds(i * BLK, BLK)] = dinv


def _tc1(x_pad, W, degp):
    return pl.pallas_call(
        _tc1_body,
        grid=(GRID,),
        in_specs=[
            pl.BlockSpec((BLK, 128), lambda i: (i, 0)),
            pl.BlockSpec((128, 128), lambda i: (0, 0)),
            pl.BlockSpec((NC, NP), lambda i: (0, 0)),
        ],
        out_specs=[
            pl.BlockSpec((BLK, 128), lambda i: (i, 0)),
            pl.BlockSpec((1, NP), lambda i: (0, 0)),
        ],
        out_shape=[
            jax.ShapeDtypeStruct((NP, 128), jnp.float32),
            jax.ShapeDtypeStruct((1, NP), jnp.float32),
        ],
    )(x_pad, W, degp)


def _make_tc_mid(F_in, F_out):
    def body(accp_ref, hs_ref, dinv_ref, b_ref, w_ref, out_ref):
        i = pl.program_id(0)
        dv = dinv_ref[0, pl.ds(i * BLK, BLK)]
        tot = accp_ref[0] + accp_ref[1] + hs_ref[...]
        o = jnp.maximum(dv[:, None] * tot + b_ref[0][None, :], 0.0)
        out_ref[...] = jnp.dot(o, w_ref[...],
                               preferred_element_type=jnp.float32) * dv[:, None]

    def run(accp, hs, dinv, b, W):
        return pl.pallas_call(
            body,
            grid=(GRID,),
            in_specs=[
                pl.BlockSpec((NC, BLK, F_in), lambda i: (0, i, 0)),
                pl.BlockSpec((BLK, F_in), lambda i: (i, 0)),
                pl.BlockSpec((1, NP), lambda i: (0, 0)),
                pl.BlockSpec((1, F_in), lambda i: (0, 0)),
                pl.BlockSpec((F_in, F_out), lambda i: (0, 0)),
            ],
            out_specs=pl.BlockSpec((BLK, F_out), lambda i: (i, 0)),
            out_shape=jax.ShapeDtypeStruct((NP, F_out), jnp.float32),
        )(accp, hs, dinv, b, W)

    return run


_tc_mid_128 = _make_tc_mid(128, 128)


def _tc_fin_body(accp_ref, hs_ref, dinv_ref, b_ref, out_ref):
    i = pl.program_id(0)
    dv = dinv_ref[0, pl.ds(i * BLK, BLK)]
    tot = accp_ref[0] + accp_ref[1] + hs_ref[...]
    o = dv[:, None] * tot + b_ref[0][None, :]
    col = lax.broadcasted_iota(jnp.int32, (BLK, 128), 1)
    valid = col < 40
    xm = jnp.where(valid, o, -jnp.inf)
    m = jnp.max(xm, axis=1)
    e = jnp.exp(xm - m[:, None])
    ssum = jnp.sum(e, axis=1)
    out_ref[...] = o - m[:, None] - jnp.log(ssum)[:, None]


def _tc_fin(accp, hs, dinv, b):
    return pl.pallas_call(
        _tc_fin_body,
        grid=(GRID,),
        in_specs=[
            pl.BlockSpec((NC, BLK, 128), lambda i: (0, i, 0)),
            pl.BlockSpec((BLK, 128), lambda i: (i, 0)),
            pl.BlockSpec((1, NP), lambda i: (0, 0)),
            pl.BlockSpec((1, 128), lambda i: (0, 0)),
        ],
        out_specs=pl.BlockSpec((BLK, 128), lambda i: (i, 0)),
        out_shape=jax.ShapeDtypeStruct((NP, 128), jnp.float32),
    )(accp, hs, dinv, b)


def kernel(x, edge_idx, W1, b1, W2, b2, W3, b3):
    src = edge_idx[0].reshape(NC, NS, NCHUNK, K)
    dst = edge_idx[1].reshape(NC, NS, NCHUNK, K)
    x_pad = jnp.pad(x, ((0, NP - N), (0, 0)))
    W3p = jnp.pad(W3, ((0, 0), (0, 128 - 40)))
    b1r = b1[None, :]
    b2r = b2[None, :]
    b3r = jnp.pad(b3, (0, 128 - 40))[None, :]

    degp = _deg_kernel(dst)

    hs1, dinv = _tc1(x_pad, W1, degp)
    acc1 = _scatter128(hs1, src, dst)
    hs2 = _tc_mid_128(acc1, hs1, dinv, b1r, W2)
    acc2 = _scatter128(hs2, src, dst)
    hs3 = _tc_mid_128(acc2, hs2, dinv, b2r, W3p)
    acc3 = _scatter128(hs3, src, dst)
    out = _tc_fin(acc3, hs3, dinv, b3r)
    return out[:N, :40]
